# baseline (speedup 1.0000x reference)
import functools

import jax
import jax.numpy as jnp
from jax import lax
from jax.experimental import pallas as pl
from jax.experimental.pallas import tpu as pltpu
from jax.experimental.pallas import tpu_sc as plsc

_TOP_RATIO = 0.3
_TOP_WEIGHT = 1.0
_IGNORE_INDEX = -1

_SPLIT = 4
_SUBR = 64
_NWIN = 4
_CHUNK = 2500


def _sc_gather(pred_flat, gt, n, c):
    info = plsc.get_sparse_core_info()
    nc, ns, lanes = info.num_cores, info.num_subcores, info.num_lanes
    nw = nc * ns
    assert n % (8 * nw) == 0
    b_per_w = n // nw
    mesh = plsc.VectorSubcoreMesh(core_axis_name="c", subcore_axis_name="s")

    @functools.partial(
        pl.kernel,
        mesh=mesh,
        out_type=jax.ShapeDtypeStruct((n,), jnp.float32),
        scratch_types=[
            pltpu.VMEM((b_per_w,), jnp.int32),
            pltpu.VMEM((b_per_w,), jnp.int32),
            pltpu.VMEM((b_per_w,), jnp.float32),
            pltpu.SemaphoreType.DMA,
        ],
    )
    def gather_k(gt_hbm, pred_hbm, out_hbm, gt_v, flat_v, vals_v, sem):
        wid = lax.axis_index("s") * nc + lax.axis_index("c")
        base = wid * b_per_w
        pltpu.sync_copy(gt_hbm.at[pl.ds(base, b_per_w)], gt_v)
        for i in range(b_per_w // lanes):
            g = gt_v[pl.ds(i * lanes, lanes)]
            g = jnp.minimum(jnp.maximum(g, 0), c - 1)
            rows = base + i * lanes + lax.iota(jnp.int32, lanes)
            flat_v[pl.ds(i * lanes, lanes)] = rows * c + g
        pltpu.async_copy(pred_hbm.at[flat_v], vals_v, sem).wait()
        pltpu.sync_copy(vals_v, out_hbm.at[pl.ds(base, b_per_w)])

    return gather_k(gt, pred_flat)


def _lse_one(pred_ref, lse_ref):
    w = pred_ref.shape[1]
    nch = w // _CHUNK
    m = jnp.max(pred_ref[:, pl.ds(0, _CHUNK)], axis=1, keepdims=True)
    for ch in range(1, nch):
        x = pred_ref[:, pl.ds(ch * _CHUNK, _CHUNK)]
        m = jnp.maximum(m, jnp.max(x, axis=1, keepdims=True))
    s = jnp.zeros_like(m)
    for ch in range(nch):
        x = pred_ref[:, pl.ds(ch * _CHUNK, _CHUNK)]
        s = s + jnp.sum(jnp.exp(x - m), axis=1, keepdims=True)
    lse_ref[...] = m + jnp.log(s)


def _lse_body(*refs):
    preds = refs[:_NWIN]
    outs = refs[_NWIN:]
    for p, o in zip(preds, outs):
        _lse_one(p, o)


def _lse_sub(pred_r):
    nr, w = pred_r.shape
    nb = nr // (_SUBR * _NWIN)
    per = nr // _NWIN
    in_specs = [
        pl.BlockSpec((_SUBR, w), functools.partial(lambda k, j: (nb * k + j, 0), k))
        for k in range(_NWIN)
    ]
    out_specs = [pl.BlockSpec((_SUBR, 1), lambda j: (j, 0)) for _ in range(_NWIN)]
    outs = pl.pallas_call(
        _lse_body,
        grid=(nb,),
        in_specs=in_specs,
        out_specs=out_specs,
        out_shape=[jax.ShapeDtypeStruct((per, 1), jnp.float32)] * _NWIN,
    )(*([pred_r] * _NWIN))
    return jnp.concatenate(outs, axis=0)


def _final_body(n, k, l0_ref, l1_ref, l2_ref, l3_ref, gat_ref, gt_ref, out_ref):
    a, b_, c_, d = l0_ref[...], l1_ref[...], l2_ref[...], l3_ref[...]
    m4 = jnp.maximum(jnp.maximum(a, b_), jnp.maximum(c_, d))
    s4 = (jnp.exp(a - m4) + jnp.exp(b_ - m4)
          + jnp.exp(c_ - m4) + jnp.exp(d - m4))
    lse = m4 + jnp.log(s4)

    nll = lse - gat_ref[...]
    valid = gt_ref[...] != _IGNORE_INDEX
    loss = jnp.where(valid, nll, jnp.float32(0.0))
    total = jnp.sum(loss)

    bb = lax.bitcast_convert_type(loss, jnp.int32)
    skey = bb ^ (lax.shift_right_arithmetic(bb, 31) & jnp.int32(0x7FFFFFFF))
    int_min = jnp.int32(-2147483648)

    def step(i, p):
        cand = p | lax.shift_left(jnp.int32(1), 31 - i)
        cnt = jnp.sum((skey >= (cand ^ int_min)).astype(jnp.int32))
        return jnp.where(cnt >= k, cand, p)

    p = lax.fori_loop(0, 32, step, jnp.int32(0))
    skey_th = p ^ int_min
    cnt_gt = jnp.sum((skey > skey_th).astype(jnp.int32))
    sum_gt = jnp.sum(jnp.where(skey > skey_th, loss, jnp.float32(0.0)))
    bits_th = skey_th ^ (lax.shift_right_arithmetic(skey_th, 31) & jnp.int32(0x7FFFFFFF))
    f_th = lax.bitcast_convert_type(bits_th, jnp.float32)
    topk_sum = sum_gt + (k - cnt_gt).astype(jnp.float32) * f_th

    out = total / jnp.float32(n) + jnp.float32(_TOP_WEIGHT) * topk_sum / jnp.float32(k)
    out_ref[...] = jnp.full((1, 1), out, jnp.float32)


def _finalize(lse_parts, gathered, gt, n, k):
    return pl.pallas_call(
        functools.partial(_final_body, n, k),
        out_shape=jax.ShapeDtypeStruct((1, 1), jnp.float32),
    )(*lse_parts, gathered, gt)


def kernel(pred, gt):
    n, c = pred.shape
    k = max(int(_TOP_RATIO * n), 1)
    w = c // _SPLIT
    gathered = _sc_gather(pred.reshape(-1), gt, n, c)
    lse_sub = _lse_sub(pred.reshape(n * _SPLIT, w))
    rows = n // 128
    l4 = lse_sub.reshape(n, _SPLIT)
    lse_parts = [l4[:, i].reshape(rows, 128) for i in range(_SPLIT)]
    out = _finalize(
        lse_parts,
        gathered.reshape(rows, 128),
        gt.reshape(rows, 128),
        n,
        k,
    )
    return out[0, 0]


# speedup vs baseline: 6.7654x; 6.7654x over previous
import functools

import jax
import jax.numpy as jnp
from jax import lax
from jax.experimental import pallas as pl
from jax.experimental.pallas import tpu as pltpu

_TOP_RATIO = 0.3
_TOP_WEIGHT = 1.0
_IGNORE_INDEX = -1

_VB = 4000
_CH = 500


def _lse_body(gt_ref, pred_ref, lsep_ref, gathp_ref):
    j = pl.program_id(0)
    vb, n = pred_ref.shape
    nch = vb // _CH

    m = jnp.max(pred_ref[pl.ds(0, _CH), :], axis=0, keepdims=True)
    for ch in range(1, nch):
        x = pred_ref[pl.ds(ch * _CH, _CH), :]
        m = jnp.maximum(m, jnp.max(x, axis=0, keepdims=True))

    target = gt_ref[...] - j * vb
    s = jnp.zeros((1, n), jnp.float32)
    g = jnp.zeros((1, n), jnp.float32)
    for ch in range(nch):
        x = pred_ref[pl.ds(ch * _CH, _CH), :]
        s = s + jnp.sum(jnp.exp(x - m), axis=0, keepdims=True)
        rows = lax.broadcasted_iota(jnp.int32, (_CH, n), 0) + ch * _CH
        g = g + jnp.sum(jnp.where(rows == target, x, jnp.float32(0.0)),
                        axis=0, keepdims=True)

    lsep_ref[...] = (m + jnp.log(s))[None]
    gathp_ref[...] = g[None]


def _lse_parts(pred_t, gt_row):
    c, n = pred_t.shape
    nb = c // _VB
    return pl.pallas_call(
        _lse_body,
        grid=(nb,),
        in_specs=[
            pl.BlockSpec((1, n), lambda j: (0, 0)),
            pl.BlockSpec((_VB, n), lambda j: (j, 0)),
        ],
        out_specs=[
            pl.BlockSpec((1, 1, n), lambda j: (j, 0, 0)),
            pl.BlockSpec((1, 1, n), lambda j: (j, 0, 0)),
        ],
        out_shape=[
            jax.ShapeDtypeStruct((nb, 1, n), jnp.float32),
            jax.ShapeDtypeStruct((nb, 1, n), jnp.float32),
        ],
    )(gt_row, pred_t)


def _final_body(n, k, lsep_ref, gathp_ref, gt_ref, out_ref):
    lsep = lsep_ref[...]
    m = jnp.max(lsep, axis=0, keepdims=True)
    s = jnp.sum(jnp.exp(lsep - m), axis=0, keepdims=True)
    lse = m + jnp.log(s)
    gat = jnp.sum(gathp_ref[...], axis=0, keepdims=True)

    nll = lse - gat
    valid = gt_ref[...] != _IGNORE_INDEX
    loss = jnp.where(valid, nll, jnp.float32(0.0))
    total = jnp.sum(loss)

    bb = lax.bitcast_convert_type(loss, jnp.int32)
    skey = bb ^ (lax.shift_right_arithmetic(bb, 31) & jnp.int32(0x7FFFFFFF))
    int_min = jnp.int32(-2147483648)

    def step(i, p):
        cand = p | lax.shift_left(jnp.int32(1), 31 - i)
        cnt = jnp.sum((skey >= (cand ^ int_min)).astype(jnp.int32))
        return jnp.where(cnt >= k, cand, p)

    p = lax.fori_loop(0, 32, step, jnp.int32(0))
    skey_th = p ^ int_min
    cnt_gt = jnp.sum((skey > skey_th).astype(jnp.int32))
    sum_gt = jnp.sum(jnp.where(skey > skey_th, loss, jnp.float32(0.0)))
    bits_th = skey_th ^ (lax.shift_right_arithmetic(skey_th, 31) & jnp.int32(0x7FFFFFFF))
    f_th = lax.bitcast_convert_type(bits_th, jnp.float32)
    topk_sum = sum_gt + (k - cnt_gt).astype(jnp.float32) * f_th

    out = total / jnp.float32(n) + jnp.float32(_TOP_WEIGHT) * topk_sum / jnp.float32(k)
    out_ref[...] = jnp.full((1, 1), out, jnp.float32)


def _finalize(lsep, gathp, gt_row, n, k):
    return pl.pallas_call(
        functools.partial(_final_body, n, k),
        out_shape=jax.ShapeDtypeStruct((1, 1), jnp.float32),
    )(lsep, gathp, gt_row)


def kernel(pred, gt):
    n, c = pred.shape
    k = max(int(_TOP_RATIO * n), 1)
    gt_row = gt.reshape(1, n)
    lsep, gathp = _lse_parts(pred.T, gt_row)
    nb = c // _VB
    out = _finalize(lsep.reshape(nb, n), gathp.reshape(nb, n), gt_row, n, k)
    return out[0, 0]


# speedup vs baseline: 6.8108x; 1.0067x over previous
import functools

import jax
import jax.numpy as jnp
from jax import lax
from jax.experimental import pallas as pl
from jax.experimental.pallas import tpu as pltpu

_TOP_RATIO = 0.3
_TOP_WEIGHT = 1.0
_IGNORE_INDEX = -1

_VB = 2000
_CH = 500


def _lse_body(gt_ref, pred_ref, lsep_ref, gathp_ref):
    j = pl.program_id(0)
    vb, n = pred_ref.shape
    nch = vb // _CH

    m = jnp.max(pred_ref[pl.ds(0, _CH), :], axis=0, keepdims=True)
    for ch in range(1, nch):
        x = pred_ref[pl.ds(ch * _CH, _CH), :]
        m = jnp.maximum(m, jnp.max(x, axis=0, keepdims=True))

    target = gt_ref[...] - j * vb
    s = jnp.zeros((1, n), jnp.float32)
    g = jnp.zeros((1, n), jnp.float32)
    for ch in range(nch):
        x = pred_ref[pl.ds(ch * _CH, _CH), :]
        s = s + jnp.sum(jnp.exp(x - m), axis=0, keepdims=True)
        rows = lax.broadcasted_iota(jnp.int32, (_CH, n), 0) + ch * _CH
        g = g + jnp.sum(jnp.where(rows == target, x, jnp.float32(0.0)),
                        axis=0, keepdims=True)

    lsep_ref[...] = (m + jnp.log(s))[None]
    gathp_ref[...] = g[None]


def _lse_parts(pred_t, gt_row):
    c, n = pred_t.shape
    nb = c // _VB
    return pl.pallas_call(
        _lse_body,
        grid=(nb,),
        in_specs=[
            pl.BlockSpec((1, n), lambda j: (0, 0)),
            pl.BlockSpec((_VB, n), lambda j: (j, 0)),
        ],
        out_specs=[
            pl.BlockSpec((1, 1, n), lambda j: (j, 0, 0)),
            pl.BlockSpec((1, 1, n), lambda j: (j, 0, 0)),
        ],
        out_shape=[
            jax.ShapeDtypeStruct((nb, 1, n), jnp.float32),
            jax.ShapeDtypeStruct((nb, 1, n), jnp.float32),
        ],
    )(gt_row, pred_t)


def _final_body(n, k, lsep_ref, gathp_ref, gt_ref, out_ref):
    lsep = lsep_ref[...]
    m = jnp.max(lsep, axis=0, keepdims=True)
    s = jnp.sum(jnp.exp(lsep - m), axis=0, keepdims=True)
    lse = m + jnp.log(s)
    gat = jnp.sum(gathp_ref[...], axis=0, keepdims=True)

    nll = lse - gat
    valid = gt_ref[...] != _IGNORE_INDEX
    loss = jnp.where(valid, nll, jnp.float32(0.0))
    total = jnp.sum(loss)

    bb = lax.bitcast_convert_type(loss, jnp.int32)
    skey = bb ^ (lax.shift_right_arithmetic(bb, 31) & jnp.int32(0x7FFFFFFF))
    int_min = jnp.int32(-2147483648)

    def step(i, p):
        cand = p | lax.shift_left(jnp.int32(1), 31 - i)
        cnt = jnp.sum((skey >= (cand ^ int_min)).astype(jnp.int32))
        return jnp.where(cnt >= k, cand, p)

    p = lax.fori_loop(0, 32, step, jnp.int32(0))
    skey_th = p ^ int_min
    cnt_gt = jnp.sum((skey > skey_th).astype(jnp.int32))
    sum_gt = jnp.sum(jnp.where(skey > skey_th, loss, jnp.float32(0.0)))
    bits_th = skey_th ^ (lax.shift_right_arithmetic(skey_th, 31) & jnp.int32(0x7FFFFFFF))
    f_th = lax.bitcast_convert_type(bits_th, jnp.float32)
    topk_sum = sum_gt + (k - cnt_gt).astype(jnp.float32) * f_th

    out = total / jnp.float32(n) + jnp.float32(_TOP_WEIGHT) * topk_sum / jnp.float32(k)
    out_ref[...] = jnp.full((1, 1), out, jnp.float32)


def _finalize(lsep, gathp, gt_row, n, k):
    return pl.pallas_call(
        functools.partial(_final_body, n, k),
        out_shape=jax.ShapeDtypeStruct((1, 1), jnp.float32),
    )(lsep, gathp, gt_row)


def kernel(pred, gt):
    n, c = pred.shape
    k = max(int(_TOP_RATIO * n), 1)
    gt_row = gt.reshape(1, n)
    lsep, gathp = _lse_parts(pred.T, gt_row)
    nb = c // _VB
    out = _finalize(lsep.reshape(nb, n), gathp.reshape(nb, n), gt_row, n, k)
    return out[0, 0]
